# baseline (speedup 1.0000x reference)
import functools

import jax
import jax.numpy as jnp
from jax import lax
from jax.experimental import pallas as pl
from jax.experimental.pallas import tpu as pltpu
from jax.experimental.pallas import tpu_sc as plsc

NUM_FIELDS = 100000
D = 64
N = 409600

NC = 2
NS = 16
NW = NC * NS
B_PER_W = N // NW
C = 256
NCHUNKS = B_PER_W // C
NPAIRS = NCHUNKS // 2
G = C // 128


def _emb_body(ids_hbm, vals_hbm, tab_hbm, out_hbm,
              idx_all, vals_all, tb0, tb1, ob0, ob1, pb,
              s_in0, s_in1, s_out0, s_out1):
    wid = lax.axis_index("s") * NC + lax.axis_index("c")
    base = wid * B_PER_W

    pltpu.sync_copy(ids_hbm.at[pl.ds(base, B_PER_W)], idx_all)
    pltpu.sync_copy(vals_hbm.at[pl.ds(base, B_PER_W)], vals_all)

    tb_b = (tb0, tb1)
    ob_b = (ob0, ob1)
    s_in = (s_in0, s_in1)
    s_out = (s_out0, s_out1)

    lanes = lax.iota(jnp.int32, 16)
    lanes16 = lanes * 16

    def gather_descs(c, slot):
        descs = []
        for j in range(G):
            off = pl.multiple_of(c * C + j * 128, 128)
            idx_ref = idx_all.at[pl.ds(off, 128)]
            dst = pl.ds(j * 128, 128)
            descs.append(pltpu.make_async_copy(
                tab_hbm.at[idx_ref], tb_b[slot].at[dst], s_in[slot]))
        return descs

    def store_desc(c, slot):
        off = pl.multiple_of(base + c * C, C)
        return pltpu.make_async_copy(
            ob_b[slot], out_hbm.at[:, pl.ds(off, C)], s_out[slot])

    def compute(c, slot):
        tb_r, ob_r = tb_b[slot], ob_b[slot]
        coff = c * C

        def group(g, _):
            r0 = g * 16
            vvec = vals_all[pl.ds(coff + r0, 16)]
            for l in range(16):
                v = vvec[l]
                skew = lanes16 + ((lanes + l) & 15)
                for k in range(D // 16):
                    fe_sl = pl.ds(k * 16, 16)
                    vs_sl = pl.ds(D + k * 16, 16)
                    res = tb_r[r0 + l, fe_sl] + v * tb_r[r0 + l, vs_sl]
                    plsc.store_scatter(pb, [skew + (256 * k)], res)
            for d in range(D):
                ridx = (16 * d) + ((lanes + d) & 15)
                row = plsc.load_gather(pb, [ridx])
                ob_r[d, pl.ds(r0, 16)] = row
            return 0

        lax.fori_loop(0, C // 16, group, 0)

    for dsc in gather_descs(0, 0):
        dsc.start()

    def pair(i, _):
        for b in (0, 1):
            c = 2 * i + b

            @pl.when(c + 1 < NCHUNKS)
            def _prefetch():
                for dsc in gather_descs(c + 1, 1 - b):
                    dsc.start()

            for dsc in gather_descs(c, b):
                dsc.wait()

            @pl.when(i >= 1)
            def _drain_store():
                store_desc(c - 2, b).wait()

            compute(c, b)
            store_desc(c, b).start()
        return 0

    lax.fori_loop(0, NPAIRS, pair, 0)
    store_desc(NCHUNKS - 2, 0).wait()
    store_desc(NCHUNKS - 1, 1).wait()


@jax.jit
def _emb_lookup(field_ids, values, table):
    mesh = plsc.VectorSubcoreMesh(
        core_axis_name="c", subcore_axis_name="s",
        num_cores=NC, num_subcores=NS)
    f = functools.partial(
        pl.kernel,
        out_type=jax.ShapeDtypeStruct((D, N), jnp.float32),
        mesh=mesh,
        compiler_params=pltpu.CompilerParams(needs_layout_passes=False),
        scratch_types=[
            pltpu.VMEM((B_PER_W,), jnp.int32),
            pltpu.VMEM((B_PER_W,), jnp.float32),
            pltpu.VMEM((C, 2 * D), jnp.float32),
            pltpu.VMEM((C, 2 * D), jnp.float32),
            pltpu.VMEM((D, C), jnp.float32),
            pltpu.VMEM((D, C), jnp.float32),
            pltpu.VMEM((16 * D,), jnp.float32),
            pltpu.SemaphoreType.DMA,
            pltpu.SemaphoreType.DMA,
            pltpu.SemaphoreType.DMA,
            pltpu.SemaphoreType.DMA,
        ],
    )(_emb_body)
    return f(field_ids, values, table)


def kernel(field_ids, values, field_embedding, value_scale):
    table = jnp.concatenate([field_embedding, value_scale], axis=1)
    out_t = _emb_lookup(field_ids.astype(jnp.int32), values, table)
    return out_t.T


# speedup vs baseline: 1.4299x; 1.4299x over previous
import functools

import jax
import jax.numpy as jnp
from jax import lax
from jax.experimental import pallas as pl
from jax.experimental.pallas import tpu as pltpu
from jax.experimental.pallas import tpu_sc as plsc

NUM_FIELDS = 100000
D = 64
N = 409600

NC = 2
NS = 16
NW = NC * NS
NSPLIT = 4
NP = N // NSPLIT
B_PER_W = NP // NW
C = 160
NCHUNKS = B_PER_W // C
NPAIRS = NCHUNKS // 2
GL = (128, 32)


def _emb_body(ids_hbm, vals_hbm, tab_hbm, out_hbm,
              idx_all, vals_all, tb0, tb1, ob0, ob1,
              s_in0, s_in1, s_out0, s_out1):
    wid = lax.axis_index("s") * NC + lax.axis_index("c")
    base = wid * B_PER_W

    pltpu.sync_copy(ids_hbm.at[pl.ds(base, B_PER_W)], idx_all)
    pltpu.sync_copy(vals_hbm.at[pl.ds(base, B_PER_W)], vals_all)

    tb_b = (tb0, tb1)
    ob_b = (ob0, ob1)
    s_in = (s_in0, s_in1)
    s_out = (s_out0, s_out1)

    def gather_descs(c, slot):
        descs = []
        j = 0
        for glen in GL:
            off = pl.multiple_of(c * C + j, 8)
            idx_ref = idx_all.at[pl.ds(off, glen)]
            dst = pl.ds(j, glen)
            descs.append(pltpu.make_async_copy(
                tab_hbm.at[idx_ref], tb_b[slot].at[dst], s_in[slot]))
            j += glen
        return descs

    def store_desc(c, slot):
        off = pl.multiple_of(base + c * C, 8)
        return pltpu.make_async_copy(
            ob_b[slot], out_hbm.at[pl.ds(off, C)], s_out[slot])

    def compute(c, slot):
        tb_r, ob_r = tb_b[slot], ob_b[slot]
        coff = c * C

        def group(g, _):
            vvec = vals_all[pl.ds(coff + g * 16, 16)]
            for rr in range(16):
                v = vvec[rr]
                r = g * 16 + rr
                for dblk in range(D // 16):
                    fe_sl = pl.ds(dblk * 16, 16)
                    vs_sl = pl.ds(D + dblk * 16, 16)
                    ob_r[r, fe_sl] = tb_r[r, fe_sl] + v * tb_r[r, vs_sl]
            return 0

        lax.fori_loop(0, C // 16, group, 0)

    for dsc in gather_descs(0, 0):
        dsc.start()

    def pair(i, _):
        for b in (0, 1):
            c = 2 * i + b

            @pl.when(c + 1 < NCHUNKS)
            def _prefetch():
                for dsc in gather_descs(c + 1, 1 - b):
                    dsc.start()

            for dsc in gather_descs(c, b):
                dsc.wait()

            @pl.when(i >= 1)
            def _drain_store():
                store_desc(c - 2, b).wait()

            compute(c, b)
            store_desc(c, b).start()
        return 0

    lax.fori_loop(0, NPAIRS, pair, 0)
    store_desc(NCHUNKS - 2, 0).wait()
    store_desc(NCHUNKS - 1, 1).wait()


@jax.jit
def _emb_lookup(field_ids, values, table):
    mesh = plsc.VectorSubcoreMesh(
        core_axis_name="c", subcore_axis_name="s",
        num_cores=NC, num_subcores=NS)
    f = functools.partial(
        pl.kernel,
        out_type=jax.ShapeDtypeStruct((NP, D), jnp.float32),
        mesh=mesh,
        scratch_types=[
            pltpu.VMEM((B_PER_W,), jnp.int32),
            pltpu.VMEM((B_PER_W,), jnp.float32),
            pltpu.VMEM((C, 2 * D), jnp.float32),
            pltpu.VMEM((C, 2 * D), jnp.float32),
            pltpu.VMEM((C, D), jnp.float32),
            pltpu.VMEM((C, D), jnp.float32),
            pltpu.SemaphoreType.DMA,
            pltpu.SemaphoreType.DMA,
            pltpu.SemaphoreType.DMA,
            pltpu.SemaphoreType.DMA,
        ],
    )(_emb_body)
    return f(field_ids, values, table)


def kernel(field_ids, values, field_embedding, value_scale):
    table = jnp.concatenate([field_embedding, value_scale], axis=1)
    ids = field_ids.astype(jnp.int32)
    out = jnp.zeros((N, D), jnp.float32)
    for s in range(NSPLIT):
        lo = s * NP
        piece = _emb_lookup(
            lax.slice(ids, (lo,), (lo + NP,)),
            lax.slice(values, (lo,), (lo + NP,)),
            table)
        out = lax.dynamic_update_slice(out, piece, (lo, 0))
    return out


# speedup vs baseline: 2.1239x; 1.4853x over previous
import functools

import jax
import jax.numpy as jnp
from jax import lax
from jax.experimental import pallas as pl
from jax.experimental.pallas import tpu as pltpu
from jax.experimental.pallas import tpu_sc as plsc

NUM_FIELDS = 100000
D = 64
N = 409600

NC = 2
NS = 16
NW = NC * NS
B_PER_W = N // NW
C = 128
NCHUNKS = B_PER_W // C
NPAIRS = NCHUNKS // 2


def _emb_body(ids_hbm, vals_hbm, tab_hbm, out_hbm,
              idx_all, vals_all, tb0, tb1, ob0, ob1,
              s_in0, s_in1, s_out0, s_out1):
    wid = lax.axis_index("s") * NC + lax.axis_index("c")
    base = wid * B_PER_W

    pltpu.sync_copy(ids_hbm.at[pl.ds(base, B_PER_W)], idx_all)
    pltpu.sync_copy(vals_hbm.at[pl.ds(base, B_PER_W)], vals_all)

    tb_b = (tb0, tb1)
    ob_b = (ob0, ob1)
    s_in = (s_in0, s_in1)
    s_out = (s_out0, s_out1)

    def gather_desc(c, slot):
        off = pl.multiple_of(c * C, C)
        idx_ref = idx_all.at[pl.ds(off, C)]
        return pltpu.make_async_copy(
            tab_hbm.at[idx_ref], tb_b[slot], s_in[slot])

    def store_desc(c, slot):
        off = pl.multiple_of(base + c * C, C)
        return pltpu.make_async_copy(
            ob_b[slot], out_hbm.at[pl.ds(off, C)], s_out[slot])

    def compute(c, slot):
        tb_r, ob_r = tb_b[slot], ob_b[slot]
        coff = c * C

        def group(g, _):
            vvec = vals_all[pl.ds(coff + g * 16, 16)]
            for rr in range(16):
                v = vvec[rr]
                r = g * 16 + rr
                for dblk in range(D // 16):
                    fe_sl = pl.ds(dblk * 16, 16)
                    vs_sl = pl.ds(D + dblk * 16, 16)
                    ob_r[r, fe_sl] = tb_r[r, fe_sl] + v * tb_r[r, vs_sl]
            return 0

        lax.fori_loop(0, C // 16, group, 0)

    gather_desc(0, 0).start()

    def pair(i, _):
        for b in (0, 1):
            c = 2 * i + b

            @pl.when(c + 1 < NCHUNKS)
            def _prefetch():
                gather_desc(c + 1, 1 - b).start()

            gather_desc(c, b).wait()

            @pl.when(i >= 1)
            def _drain_store():
                store_desc(c - 2, b).wait()

            compute(c, b)
            store_desc(c, b).start()
        return 0

    lax.fori_loop(0, NPAIRS, pair, 0)
    store_desc(NCHUNKS - 2, 0).wait()
    store_desc(NCHUNKS - 1, 1).wait()


@jax.jit
def _emb_lookup(field_ids, values, table):
    mesh = plsc.VectorSubcoreMesh(
        core_axis_name="c", subcore_axis_name="s",
        num_cores=NC, num_subcores=NS)
    f = functools.partial(
        pl.kernel,
        out_type=jax.ShapeDtypeStruct((N, D), jnp.float32),
        mesh=mesh,
        scratch_types=[
            pltpu.VMEM((B_PER_W,), jnp.int32),
            pltpu.VMEM((B_PER_W,), jnp.float32),
            pltpu.VMEM((C, 2 * D), jnp.float32),
            pltpu.VMEM((C, 2 * D), jnp.float32),
            pltpu.VMEM((C, D), jnp.float32),
            pltpu.VMEM((C, D), jnp.float32),
            pltpu.SemaphoreType.DMA,
            pltpu.SemaphoreType.DMA,
            pltpu.SemaphoreType.DMA,
            pltpu.SemaphoreType.DMA,
        ],
    )(_emb_body)
    return f(field_ids, values, table)


def kernel(field_ids, values, field_embedding, value_scale):
    table = jnp.concatenate([field_embedding, value_scale], axis=1)
    out = _emb_lookup(field_ids.astype(jnp.int32), values, table)
    return lax.optimization_barrier(out)


# speedup vs baseline: 2.1519x; 1.0132x over previous
import functools

import jax
import jax.numpy as jnp
from jax import lax
from jax.experimental import pallas as pl
from jax.experimental.pallas import tpu as pltpu
from jax.experimental.pallas import tpu_sc as plsc

NUM_FIELDS = 100000
D = 64
N = 409600

NC = 2
NS = 16
NW = NC * NS
B_PER_W = N // NW
C = 160
NCHUNKS = B_PER_W // C
NPAIRS = NCHUNKS // 2
GL = (128, 32)


def _emb_body(ids_hbm, vals_hbm, tab_hbm, out_hbm,
              idx_all, vals_all, tb0, tb1, ob0, ob1,
              s_in0, s_in1, s_out0, s_out1):
    wid = lax.axis_index("s") * NC + lax.axis_index("c")
    base = wid * B_PER_W

    pltpu.sync_copy(ids_hbm.at[pl.ds(base, B_PER_W)], idx_all)
    pltpu.sync_copy(vals_hbm.at[pl.ds(base, B_PER_W)], vals_all)

    tb_b = (tb0, tb1)
    ob_b = (ob0, ob1)
    s_in = (s_in0, s_in1)
    s_out = (s_out0, s_out1)

    def gather_descs(c, slot):
        descs = []
        j = 0
        for glen in GL:
            off = pl.multiple_of(c * C + j, 8)
            idx_ref = idx_all.at[pl.ds(off, glen)]
            dst = pl.ds(j, glen)
            descs.append(pltpu.make_async_copy(
                tab_hbm.at[idx_ref], tb_b[slot].at[dst], s_in[slot]))
            j += glen
        return descs

    def store_desc(c, slot):
        off = pl.multiple_of(base + c * C, 8)
        return pltpu.make_async_copy(
            ob_b[slot], out_hbm.at[pl.ds(off, C)], s_out[slot])

    def compute(c, slot):
        tb_r, ob_r = tb_b[slot], ob_b[slot]
        coff = c * C

        def group(g, _):
            vvec = vals_all[pl.ds(coff + g * 16, 16)]
            for rr in range(16):
                v = vvec[rr]
                r = g * 16 + rr
                for dblk in range(D // 16):
                    fe_sl = pl.ds(dblk * 16, 16)
                    vs_sl = pl.ds(D + dblk * 16, 16)
                    ob_r[r, fe_sl] = tb_r[r, fe_sl] + v * tb_r[r, vs_sl]
            return 0

        lax.fori_loop(0, C // 16, group, 0)

    for dsc in gather_descs(0, 0):
        dsc.start()

    def pair(i, _):
        for b in (0, 1):
            c = 2 * i + b

            @pl.when(c + 1 < NCHUNKS)
            def _prefetch():
                for dsc in gather_descs(c + 1, 1 - b):
                    dsc.start()

            for dsc in gather_descs(c, b):
                dsc.wait()

            @pl.when(i >= 1)
            def _drain_store():
                store_desc(c - 2, b).wait()

            compute(c, b)
            store_desc(c, b).start()
        return 0

    lax.fori_loop(0, NPAIRS, pair, 0)
    store_desc(NCHUNKS - 2, 0).wait()
    store_desc(NCHUNKS - 1, 1).wait()


@jax.jit
def _emb_lookup(field_ids, values, table):
    mesh = plsc.VectorSubcoreMesh(
        core_axis_name="c", subcore_axis_name="s",
        num_cores=NC, num_subcores=NS)
    f = functools.partial(
        pl.kernel,
        out_type=jax.ShapeDtypeStruct((N, D), jnp.float32),
        mesh=mesh,
        scratch_types=[
            pltpu.VMEM((B_PER_W,), jnp.int32),
            pltpu.VMEM((B_PER_W,), jnp.float32),
            pltpu.VMEM((C, 2 * D), jnp.float32),
            pltpu.VMEM((C, 2 * D), jnp.float32),
            pltpu.VMEM((C, D), jnp.float32),
            pltpu.VMEM((C, D), jnp.float32),
            pltpu.SemaphoreType.DMA,
            pltpu.SemaphoreType.DMA,
            pltpu.SemaphoreType.DMA,
            pltpu.SemaphoreType.DMA,
        ],
    )(_emb_body)
    return f(field_ids, values, table)


def kernel(field_ids, values, field_embedding, value_scale):
    table = jnp.concatenate([field_embedding, value_scale], axis=1)
    out = _emb_lookup(field_ids.astype(jnp.int32), values, table)
    return lax.optimization_barrier(out)
